# baseline (speedup 1.0000x reference)
import functools

import jax
import jax.numpy as jnp
from jax import lax
from jax.experimental import pallas as pl
from jax.experimental.pallas import tpu as pltpu
from jax.experimental.pallas import tpu_sc as plsc

_N = 10000
_D = 128
_E = 320000
_ETOT = _E + _N
_NEG = 0.2

_NC = 2
_NS = 16
_L = 16
_NW = _NC * _NS
_CE = 64
_CHUNKS = -(-_ETOT // (_NW * _CE * 2)) * 2
_EW = _CHUNKS * _CE
_EPAD = _EW * _NW
_NP = 10240
_RPS = _NP // _NS
_RZ = 64


def _build_edge_pass():
    mesh = plsc.VectorSubcoreMesh(core_axis_name="c", subcore_axis_name="s")

    @functools.partial(
        pl.kernel,
        out_type=(
            jax.ShapeDtypeStruct((_NC, _NP, _D), jnp.float32),
            jax.ShapeDtypeStruct((_NC, _NP, _L), jnp.float32),
        ),
        mesh=mesh,
        compiler_params=pltpu.CompilerParams(
            needs_layout_passes=False, use_tc_tiling_on_sc=False),
        scratch_types=[
            pltpu.VMEM_SHARED((_NP, _D), jnp.float32),
            pltpu.VMEM_SHARED((_NP, _L), jnp.float32),
            pltpu.VMEM((_CE,), jnp.int32),
            pltpu.VMEM((_CE,), jnp.int32),
            pltpu.VMEM((_CE,), jnp.int32),
            pltpu.VMEM((_CE,), jnp.int32),
            pltpu.VMEM((_CE, _D), jnp.float32),
            pltpu.VMEM((_CE, _D), jnp.float32),
            pltpu.VMEM((_CE, _D), jnp.float32),
            pltpu.VMEM((_CE, _D), jnp.float32),
            pltpu.VMEM((_CE, _L), jnp.float32),
            pltpu.VMEM((_CE, _L), jnp.float32),
            pltpu.VMEM((_D,), jnp.float32),
            pltpu.SemaphoreType.DMA,
            pltpu.SemaphoreType.DMA,
            pltpu.SemaphoreType.DMA,
            pltpu.SemaphoreType.DMA,
            pltpu.SemaphoreType.DMA,
            pltpu.SemaphoreType.DMA,
        ],
    )
    def edge_kernel(xl_hbm, xr_hbm, src_hbm, dst_hbm, att_hbm,
                    num_out, den_out,
                    num_sh, den_sh, srcv0, srcv1, dstv0, dstv1,
                    xlv0, xlv1, xrv0, xrv1, sv0, sv1, attv,
                    semi0, semi1, semd0, semd1, semsc0, semsc1):
        cid = lax.axis_index("c")
        sid = lax.axis_index("s")
        wid = sid * _NC + cid
        lanes = lax.iota(jnp.int32, _L)
        zero16 = jnp.zeros((_L,), jnp.float32)
        zero16i = jnp.zeros((_L,), jnp.int32)
        srcvs = (srcv0, srcv1)
        dstvs = (dstv0, dstv1)
        xlvs = (xlv0, xlv1)
        xrvs = (xrv0, xrv1)
        svs = (sv0, sv1)
        semis = (semi0, semi1)
        semds = (semd0, semd1)
        semscs = (semsc0, semsc1)

        def _zn(i, carry):
            for k in range(_D // _L):
                xlv0[i, pl.ds(k * _L, _L)] = zero16
            return carry
        lax.fori_loop(0, _RZ, _zn, 0)

        def _zs(i, carry):
            sv0[i, :] = zero16
            return carry
        lax.fori_loop(0, _CE, _zs, 0)

        row0 = sid * _RPS

        def _zinit(j, carry):
            r = row0 + j * _RZ
            pltpu.sync_copy(xlv0, num_sh.at[pl.ds(r, _RZ)])
            pltpu.sync_copy(sv0, den_sh.at[pl.ds(r, _RZ)])
            return carry
        lax.fori_loop(0, _RPS // _RZ, _zinit, 0)
        pltpu.sync_copy(att_hbm, attv)
        plsc.subcore_barrier()

        base0 = wid * _EW

        def _start_idx(t, b):
            off = base0 + t * _CE
            return (
                pltpu.async_copy(src_hbm.at[pl.ds(off, _CE)], srcvs[b],
                                 semis[b]),
                pltpu.async_copy(dst_hbm.at[pl.ds(off, _CE)], dstvs[b],
                                 semis[b]),
            )

        def _wait_idx(b):
            pltpu.make_async_copy(src_hbm.at[pl.ds(0, _CE)], srcvs[b],
                                  semis[b]).wait()
            pltpu.make_async_copy(dst_hbm.at[pl.ds(0, _CE)], dstvs[b],
                                  semis[b]).wait()

        def _start_rows(b):
            pltpu.async_copy(xl_hbm.at[srcvs[b]], xlvs[b], semds[b])
            pltpu.async_copy(xr_hbm.at[dstvs[b]], xrvs[b], semds[b])

        def _wait_rows(b):
            pltpu.make_async_copy(xl_hbm.at[srcvs[b]], xlvs[b],
                                  semds[b]).wait()
            pltpu.make_async_copy(xr_hbm.at[dstvs[b]], xrvs[b],
                                  semds[b]).wait()

        _start_idx(0, 0)
        _wait_idx(0)
        _start_rows(0)
        _start_idx(1, 1)

        ngrp = _CE // _L
        def _compute(t, b, attks):
            xlv = xlvs[b]
            xrv = xrvs[b]
            sv = svs[b]
            base = base0 + t * _CE

            def _edge(e, masked):
                parts = []
                for k in range(_D // _L):
                    z = (xlv[e, pl.ds(k * _L, _L)]
                         + xrv[e, pl.ds(k * _L, _L)])
                    m = jnp.maximum(z, _NEG * z)
                    parts.append(m * attks[k])
                while len(parts) > 1:
                    parts = [parts[i] + parts[i + 1]
                             for i in range(0, len(parts), 2)]
                ev = jnp.exp(parts[0])
                if masked:
                    valid = base + e < _ETOT
                    sv[e, :] = jnp.where(
                        jnp.logical_and(lanes == 0, valid), ev, zero16)
                    sev = jnp.where(valid, ev, zero16)
                else:
                    sv[e, :] = jnp.where(lanes == 0, ev, zero16)
                    sev = ev
                for k in range(_D // _L):
                    xlv[e, pl.ds(k * _L, _L)] = (
                        xlv[e, pl.ds(k * _L, _L)] * sev)

            def _eloop(masked):
                @plsc.parallel_loop(0, _CE, 1, unroll=2)
                def _(e):
                    _edge(e, masked)

            @pl.when(base + _CE <= _ETOT)
            def _():
                _eloop(False)

            @pl.when(base + _CE > _ETOT)
            def _():
                _eloop(True)

            pltpu.async_copy(xlv, num_sh.at[dstvs[b]], semscs[b], add=True)
            pltpu.async_copy(sv, den_sh.at[dstvs[b]], semscs[b], add=True)

        def _wait_scatter(b):
            pltpu.make_async_copy(xlvs[b], num_sh.at[dstvs[b]],
                                  semscs[b]).wait()
            pltpu.make_async_copy(svs[b], den_sh.at[dstvs[b]],
                                  semscs[b]).wait()

        attks = [attv[pl.ds(k * _L, _L)] for k in range(_D // _L)]

        def pair_body(tt, carry):
            for b in range(2):
                t = tt * 2 + b
                nb = 1 - b

                @pl.when(t + 1 < _CHUNKS)
                def _():
                    _wait_idx(nb)

                    @pl.when(t >= 1)
                    def _():
                        _wait_scatter(nb)

                    _start_rows(nb)

                _wait_rows(b)
                _compute(t, b, attks)

                @pl.when(t + 2 < _CHUNKS)
                def _():
                    _start_idx(t + 2, b)
            return carry

        lax.fori_loop(0, _CHUNKS // 2, pair_body, 0)
        _wait_scatter(0)
        _wait_scatter(1)
        plsc.subcore_barrier()

        def _wb(j, carry):
            r = row0 + j * _RZ
            pltpu.sync_copy(num_sh.at[pl.ds(r, _RZ)], xlv0)
            pltpu.sync_copy(xlv0, num_out.at[cid, pl.ds(r, _RZ)])
            pltpu.sync_copy(den_sh.at[pl.ds(r, _RZ)], sv0)
            pltpu.sync_copy(sv0, den_out.at[cid, pl.ds(r, _RZ)])
            return carry
        lax.fori_loop(0, _RPS // _RZ, _wb, 0)

    return edge_kernel


_edge_pass = _build_edge_pass()


_BR = 1000


def _mm1_body(x_ref, wl_ref, bl_ref, wr_ref, br_ref, xl_ref, xr_ref):
    xb = x_ref[...]
    xl_ref[...] = jnp.dot(xb, wl_ref[...], precision=lax.Precision.HIGHEST,
                          preferred_element_type=jnp.float32) + bl_ref[...]
    xr_ref[...] = jnp.dot(xb, wr_ref[...], precision=lax.Precision.HIGHEST,
                          preferred_element_type=jnp.float32) + br_ref[...]


def _proj1(x, wl, bl, wr, br):
    return pl.pallas_call(
        _mm1_body,
        grid=(_N // _BR,),
        in_specs=[
            pl.BlockSpec((_BR, _D), lambda i: (i, 0)),
            pl.BlockSpec((_D, _D), lambda i: (0, 0)),
            pl.BlockSpec((1, _D), lambda i: (0, 0)),
            pl.BlockSpec((_D, _D), lambda i: (0, 0)),
            pl.BlockSpec((1, _D), lambda i: (0, 0)),
        ],
        out_specs=[pl.BlockSpec((_BR, _D), lambda i: (i, 0)),
                   pl.BlockSpec((_BR, _D), lambda i: (i, 0))],
        out_shape=[jax.ShapeDtypeStruct((_N, _D), jnp.float32)] * 2,
    )(x, wl, bl.reshape(1, _D), wr, br.reshape(1, _D))


def _combine_mm_body(n0_ref, n1_ref, d0_ref, d1_ref, bias_ref,
                     wl_ref, bl_ref, wr_ref, br_ref, xl_ref, xr_ref):
    den = d0_ref[...] + d1_ref[...] + 1e-16
    h = (n0_ref[...] + n1_ref[...]) / den + bias_ref[...]
    h = jnp.maximum(h, 0.0)
    xl_ref[...] = jnp.dot(h, wl_ref[...], precision=lax.Precision.HIGHEST,
                          preferred_element_type=jnp.float32) + bl_ref[...]
    xr_ref[...] = jnp.dot(h, wr_ref[...], precision=lax.Precision.HIGHEST,
                          preferred_element_type=jnp.float32) + br_ref[...]


def _proj2(n0, n1, d0, d1, bias, wl, bl, wr, br):
    return pl.pallas_call(
        _combine_mm_body,
        grid=(_N // _BR,),
        in_specs=[
            pl.BlockSpec((_BR, _D), lambda i: (i, 0)),
            pl.BlockSpec((_BR, _D), lambda i: (i, 0)),
            pl.BlockSpec((_BR, 1), lambda i: (i, 0)),
            pl.BlockSpec((_BR, 1), lambda i: (i, 0)),
            pl.BlockSpec((1, _D), lambda i: (0, 0)),
            pl.BlockSpec((_D, _D), lambda i: (0, 0)),
            pl.BlockSpec((1, _D), lambda i: (0, 0)),
            pl.BlockSpec((_D, _D), lambda i: (0, 0)),
            pl.BlockSpec((1, _D), lambda i: (0, 0)),
        ],
        out_specs=[pl.BlockSpec((_BR, _D), lambda i: (i, 0)),
                   pl.BlockSpec((_BR, _D), lambda i: (i, 0))],
        out_shape=[jax.ShapeDtypeStruct((_N, _D), jnp.float32)] * 2,
    )(n0, n1, d0, d1, bias.reshape(1, _D), wl, bl.reshape(1, _D),
      wr, br.reshape(1, _D))


def _final_body(n0_ref, n1_ref, d0_ref, d1_ref, bias_ref, out_ref):
    den = d0_ref[...] + d1_ref[...] + 1e-16
    h = (n0_ref[...] + n1_ref[...]) / den + bias_ref[...]
    out_ref[...] = jnp.maximum(h, 0.0)


def _final(n0, n1, d0, d1, bias):
    return pl.pallas_call(
        _final_body,
        grid=(_N // _BR,),
        in_specs=[
            pl.BlockSpec((_BR, _D), lambda i: (i, 0)),
            pl.BlockSpec((_BR, _D), lambda i: (i, 0)),
            pl.BlockSpec((_BR, 1), lambda i: (i, 0)),
            pl.BlockSpec((_BR, 1), lambda i: (i, 0)),
            pl.BlockSpec((1, _D), lambda i: (0, 0)),
        ],
        out_specs=pl.BlockSpec((_BR, _D), lambda i: (i, 0)),
        out_shape=jax.ShapeDtypeStruct((_N, _D), jnp.float32),
    )(n0, n1, d0, d1, bias.reshape(1, _D))


def kernel(x, edge_index, W_l1, b_l1, W_r1, b_r1, att1, bias1,
           W_l2, b_l2, W_r2, b_r2, att2, bias2):
    idt = edge_index.dtype
    loop = jnp.arange(_N, dtype=idt)
    padz = jnp.zeros((_EPAD - _ETOT,), dtype=idt)
    src = jnp.concatenate([edge_index[0], loop, padz])
    dst = jnp.concatenate([edge_index[1], loop, padz])

    xl1, xr1 = _proj1(x, W_l1, b_l1, W_r1, b_r1)
    num1, den1 = _edge_pass(xl1, xr1, src, dst, att1.reshape(_D))
    xl2, xr2 = _proj2(num1[0, :_N], num1[1, :_N],
                      den1[0, :_N, 0:1], den1[1, :_N, 0:1],
                      bias1, W_l2, b_l2, W_r2, b_r2)
    num2, den2 = _edge_pass(xl2, xr2, src, dst, att2.reshape(_D))
    return _final(num2[0, :_N], num2[1, :_N],
                  den2[0, :_N, 0:1], den2[1, :_N, 0:1], bias2)


# speedup vs baseline: 1.0143x; 1.0143x over previous
import functools

import jax
import jax.numpy as jnp
from jax import lax
from jax.experimental import pallas as pl
from jax.experimental.pallas import tpu as pltpu
from jax.experimental.pallas import tpu_sc as plsc

_N = 10000
_D = 128
_E = 320000
_ETOT = _E + _N
_NEG = 0.2

_NC = 2
_NS = 16
_L = 16
_NW = _NC * _NS
_CE = 64
_CHUNKS = -(-_ETOT // (_NW * _CE * 2)) * 2
_EW = _CHUNKS * _CE
_EPAD = _EW * _NW
_NP = 10240
_RPS = _NP // _NS
_RZ = 64


def _build_edge_pass():
    mesh = plsc.VectorSubcoreMesh(core_axis_name="c", subcore_axis_name="s")

    @functools.partial(
        pl.kernel,
        out_type=(
            jax.ShapeDtypeStruct((_NC, _NP, _D), jnp.float32),
            jax.ShapeDtypeStruct((_NC, _NP, _L), jnp.float32),
        ),
        mesh=mesh,
        compiler_params=pltpu.CompilerParams(
            needs_layout_passes=False, use_tc_tiling_on_sc=False),
        scratch_types=[
            pltpu.VMEM_SHARED((_NP, _D), jnp.float32),
            pltpu.VMEM_SHARED((_NP, _L), jnp.float32),
            pltpu.VMEM((_CE,), jnp.int32),
            pltpu.VMEM((_CE,), jnp.int32),
            pltpu.VMEM((_CE,), jnp.int32),
            pltpu.VMEM((_CE,), jnp.int32),
            pltpu.VMEM((_CE, _D), jnp.float32),
            pltpu.VMEM((_CE, _D), jnp.float32),
            pltpu.VMEM((_CE, _D), jnp.float32),
            pltpu.VMEM((_CE, _D), jnp.float32),
            pltpu.VMEM((_CE, _L), jnp.float32),
            pltpu.VMEM((_CE, _L), jnp.float32),
            pltpu.VMEM((_D,), jnp.float32),
            pltpu.SemaphoreType.DMA,
            pltpu.SemaphoreType.DMA,
            pltpu.SemaphoreType.DMA,
            pltpu.SemaphoreType.DMA,
            pltpu.SemaphoreType.DMA,
            pltpu.SemaphoreType.DMA,
        ],
    )
    def edge_kernel(xl_hbm, xr_hbm, src_hbm, dst_hbm, att_hbm,
                    num_out, den_out,
                    num_sh, den_sh, srcv0, srcv1, dstv0, dstv1,
                    xlv0, xlv1, xrv0, xrv1, sv0, sv1, attv,
                    semi0, semi1, semd0, semd1, semsc0, semsc1):
        cid = lax.axis_index("c")
        sid = lax.axis_index("s")
        wid = sid * _NC + cid
        lanes = lax.iota(jnp.int32, _L)
        zero16 = jnp.zeros((_L,), jnp.float32)
        zero16i = jnp.zeros((_L,), jnp.int32)
        srcvs = (srcv0, srcv1)
        dstvs = (dstv0, dstv1)
        xlvs = (xlv0, xlv1)
        xrvs = (xrv0, xrv1)
        svs = (sv0, sv1)
        semis = (semi0, semi1)
        semds = (semd0, semd1)
        semscs = (semsc0, semsc1)

        def _zn(i, carry):
            for k in range(_D // _L):
                xlv0[i, pl.ds(k * _L, _L)] = zero16
            return carry
        lax.fori_loop(0, _RZ, _zn, 0)

        def _zs(i, carry):
            sv0[i, :] = zero16
            return carry
        lax.fori_loop(0, _CE, _zs, 0)

        row0 = sid * _RPS

        def _zinit(j, carry):
            r = row0 + j * _RZ
            pltpu.sync_copy(xlv0, num_sh.at[pl.ds(r, _RZ)])
            pltpu.sync_copy(sv0, den_sh.at[pl.ds(r, _RZ)])
            return carry
        lax.fori_loop(0, _RPS // _RZ, _zinit, 0)
        pltpu.sync_copy(att_hbm, attv)
        plsc.subcore_barrier()

        base0 = wid * _EW

        def _start_idx(t, b):
            off = base0 + t * _CE
            return (
                pltpu.async_copy(src_hbm.at[pl.ds(off, _CE)], srcvs[b],
                                 semis[b]),
                pltpu.async_copy(dst_hbm.at[pl.ds(off, _CE)], dstvs[b],
                                 semis[b]),
            )

        def _wait_idx(b):
            pltpu.make_async_copy(src_hbm.at[pl.ds(0, _CE)], srcvs[b],
                                  semis[b]).wait()
            pltpu.make_async_copy(dst_hbm.at[pl.ds(0, _CE)], dstvs[b],
                                  semis[b]).wait()

        def _start_rows(b):
            pltpu.async_copy(xl_hbm.at[srcvs[b]], xlvs[b], semds[b])
            pltpu.async_copy(xr_hbm.at[dstvs[b]], xrvs[b], semds[b])

        def _wait_rows(b):
            pltpu.make_async_copy(xl_hbm.at[srcvs[b]], xlvs[b],
                                  semds[b]).wait()
            pltpu.make_async_copy(xr_hbm.at[dstvs[b]], xrvs[b],
                                  semds[b]).wait()

        _start_idx(0, 0)
        _wait_idx(0)
        _start_rows(0)
        _start_idx(1, 1)

        ngrp = _CE // _L
        def _compute(t, b, attks):
            xlv = xlvs[b]
            xrv = xrvs[b]
            sv = svs[b]
            base = base0 + t * _CE

            def _edge(e, masked):
                parts = []
                for k in range(_D // _L):
                    z = (xlv[e, pl.ds(k * _L, _L)]
                         + xrv[e, pl.ds(k * _L, _L)])
                    m = jnp.maximum(z, _NEG * z)
                    parts.append(m * attks[k])
                while len(parts) > 1:
                    parts = [parts[i] + parts[i + 1]
                             for i in range(0, len(parts), 2)]
                alpha = jnp.sum(parts[0])
                ev = jnp.exp(jnp.full((_L,), alpha, jnp.float32))
                if masked:
                    valid = base + e < _ETOT
                    sv[e, :] = jnp.where(
                        jnp.logical_and(lanes == 0, valid), ev, zero16)
                    sev = jnp.where(valid, ev, zero16)
                else:
                    sv[e, :] = jnp.where(lanes == 0, ev, zero16)
                    sev = ev
                for k in range(_D // _L):
                    xlv[e, pl.ds(k * _L, _L)] = (
                        xlv[e, pl.ds(k * _L, _L)] * sev)

            def _eloop(masked):
                @plsc.parallel_loop(0, _CE, 1, unroll=2)
                def _(e):
                    _edge(e, masked)

            @pl.when(base + _CE <= _ETOT)
            def _():
                _eloop(False)

            @pl.when(base + _CE > _ETOT)
            def _():
                _eloop(True)

            pltpu.async_copy(xlv, num_sh.at[dstvs[b]], semscs[b], add=True)
            pltpu.async_copy(sv, den_sh.at[dstvs[b]], semscs[b], add=True)

        def _wait_scatter(b):
            pltpu.make_async_copy(xlvs[b], num_sh.at[dstvs[b]],
                                  semscs[b]).wait()
            pltpu.make_async_copy(svs[b], den_sh.at[dstvs[b]],
                                  semscs[b]).wait()

        attks = [attv[pl.ds(k * _L, _L)] for k in range(_D // _L)]

        def pair_body(tt, carry):
            for b in range(2):
                t = tt * 2 + b
                nb = 1 - b

                @pl.when(t + 1 < _CHUNKS)
                def _():
                    _wait_idx(nb)

                    @pl.when(t >= 1)
                    def _():
                        _wait_scatter(nb)

                    _start_rows(nb)

                _wait_rows(b)
                _compute(t, b, attks)

                @pl.when(t + 2 < _CHUNKS)
                def _():
                    _start_idx(t + 2, b)
            return carry

        lax.fori_loop(0, _CHUNKS // 2, pair_body, 0)
        _wait_scatter(0)
        _wait_scatter(1)
        plsc.subcore_barrier()

        def _wb(j, carry):
            r = row0 + j * _RZ
            pltpu.sync_copy(num_sh.at[pl.ds(r, _RZ)], xlv0)
            pltpu.sync_copy(xlv0, num_out.at[cid, pl.ds(r, _RZ)])
            pltpu.sync_copy(den_sh.at[pl.ds(r, _RZ)], sv0)
            pltpu.sync_copy(sv0, den_out.at[cid, pl.ds(r, _RZ)])
            return carry
        lax.fori_loop(0, _RPS // _RZ, _wb, 0)

    return edge_kernel


_edge_pass = _build_edge_pass()


_BR = 1000


def _mm1_body(x_ref, wl_ref, bl_ref, wr_ref, br_ref, xl_ref, xr_ref):
    xb = x_ref[...]
    xl_ref[...] = jnp.dot(xb, wl_ref[...], precision=lax.Precision.HIGHEST,
                          preferred_element_type=jnp.float32) + bl_ref[...]
    xr_ref[...] = jnp.dot(xb, wr_ref[...], precision=lax.Precision.HIGHEST,
                          preferred_element_type=jnp.float32) + br_ref[...]


def _proj1(x, wl, bl, wr, br):
    return pl.pallas_call(
        _mm1_body,
        grid=(_N // _BR,),
        in_specs=[
            pl.BlockSpec((_BR, _D), lambda i: (i, 0)),
            pl.BlockSpec((_D, _D), lambda i: (0, 0)),
            pl.BlockSpec((1, _D), lambda i: (0, 0)),
            pl.BlockSpec((_D, _D), lambda i: (0, 0)),
            pl.BlockSpec((1, _D), lambda i: (0, 0)),
        ],
        out_specs=[pl.BlockSpec((_BR, _D), lambda i: (i, 0)),
                   pl.BlockSpec((_BR, _D), lambda i: (i, 0))],
        out_shape=[jax.ShapeDtypeStruct((_N, _D), jnp.float32)] * 2,
    )(x, wl, bl.reshape(1, _D), wr, br.reshape(1, _D))


def _combine_mm_body(n0_ref, n1_ref, d0_ref, d1_ref, bias_ref,
                     wl_ref, bl_ref, wr_ref, br_ref, xl_ref, xr_ref):
    den = d0_ref[...] + d1_ref[...] + 1e-16
    h = (n0_ref[...] + n1_ref[...]) / den + bias_ref[...]
    h = jnp.maximum(h, 0.0)
    xl_ref[...] = jnp.dot(h, wl_ref[...], precision=lax.Precision.HIGHEST,
                          preferred_element_type=jnp.float32) + bl_ref[...]
    xr_ref[...] = jnp.dot(h, wr_ref[...], precision=lax.Precision.HIGHEST,
                          preferred_element_type=jnp.float32) + br_ref[...]


def _proj2(n0, n1, d0, d1, bias, wl, bl, wr, br):
    return pl.pallas_call(
        _combine_mm_body,
        grid=(_N // _BR,),
        in_specs=[
            pl.BlockSpec((_BR, _D), lambda i: (i, 0)),
            pl.BlockSpec((_BR, _D), lambda i: (i, 0)),
            pl.BlockSpec((_BR, 1), lambda i: (i, 0)),
            pl.BlockSpec((_BR, 1), lambda i: (i, 0)),
            pl.BlockSpec((1, _D), lambda i: (0, 0)),
            pl.BlockSpec((_D, _D), lambda i: (0, 0)),
            pl.BlockSpec((1, _D), lambda i: (0, 0)),
            pl.BlockSpec((_D, _D), lambda i: (0, 0)),
            pl.BlockSpec((1, _D), lambda i: (0, 0)),
        ],
        out_specs=[pl.BlockSpec((_BR, _D), lambda i: (i, 0)),
                   pl.BlockSpec((_BR, _D), lambda i: (i, 0))],
        out_shape=[jax.ShapeDtypeStruct((_N, _D), jnp.float32)] * 2,
    )(n0, n1, d0, d1, bias.reshape(1, _D), wl, bl.reshape(1, _D),
      wr, br.reshape(1, _D))


def _final_body(n0_ref, n1_ref, d0_ref, d1_ref, bias_ref, out_ref):
    den = d0_ref[...] + d1_ref[...] + 1e-16
    h = (n0_ref[...] + n1_ref[...]) / den + bias_ref[...]
    out_ref[...] = jnp.maximum(h, 0.0)


def _final(n0, n1, d0, d1, bias):
    return pl.pallas_call(
        _final_body,
        grid=(_N // _BR,),
        in_specs=[
            pl.BlockSpec((_BR, _D), lambda i: (i, 0)),
            pl.BlockSpec((_BR, _D), lambda i: (i, 0)),
            pl.BlockSpec((_BR, 1), lambda i: (i, 0)),
            pl.BlockSpec((_BR, 1), lambda i: (i, 0)),
            pl.BlockSpec((1, _D), lambda i: (0, 0)),
        ],
        out_specs=pl.BlockSpec((_BR, _D), lambda i: (i, 0)),
        out_shape=jax.ShapeDtypeStruct((_N, _D), jnp.float32),
    )(n0, n1, d0, d1, bias.reshape(1, _D))


def kernel(x, edge_index, W_l1, b_l1, W_r1, b_r1, att1, bias1,
           W_l2, b_l2, W_r2, b_r2, att2, bias2):
    idt = edge_index.dtype
    loop = jnp.arange(_N, dtype=idt)
    padz = jnp.zeros((_EPAD - _ETOT,), dtype=idt)
    src = jnp.concatenate([edge_index[0], loop, padz])
    dst = jnp.concatenate([edge_index[1], loop, padz])

    xl1, xr1 = _proj1(x, W_l1, b_l1, W_r1, b_r1)
    num1, den1 = _edge_pass(xl1, xr1, src, dst, att1.reshape(_D))
    xl2, xr2 = _proj2(num1[0, :_N], num1[1, :_N],
                      den1[0, :_N, 0:1], den1[1, :_N, 0:1],
                      bias1, W_l2, b_l2, W_r2, b_r2)
    num2, den2 = _edge_pass(xl2, xr2, src, dst, att2.reshape(_D))
    return _final(num2[0, :_N], num2[1, :_N],
                  den2[0, :_N, 0:1], den2[1, :_N, 0:1], bias2)


# speedup vs baseline: 1.0633x; 1.0483x over previous
import functools

import jax
import jax.numpy as jnp
from jax import lax
from jax.experimental import pallas as pl
from jax.experimental.pallas import tpu as pltpu
from jax.experimental.pallas import tpu_sc as plsc

_N = 10000
_D = 128
_E = 320000
_ETOT = _E + _N
_NEG = 0.2

_NC = 2
_NS = 16
_L = 16
_NW = _NC * _NS
_CE = 64
_CHUNKS = -(-_ETOT // (_NW * _CE * 2)) * 2
_EW = _CHUNKS * _CE
_EPAD = _EW * _NW
_NP = 10240
_RPS = _NP // _NS
_RZ = 64


def _build_edge_pass():
    mesh = plsc.VectorSubcoreMesh(core_axis_name="c", subcore_axis_name="s")

    @functools.partial(
        pl.kernel,
        out_type=(
            jax.ShapeDtypeStruct((_NC, _NP, _D), jnp.float32),
            jax.ShapeDtypeStruct((_NC, _NP, _L), jnp.float32),
        ),
        mesh=mesh,
        compiler_params=pltpu.CompilerParams(
            needs_layout_passes=False, use_tc_tiling_on_sc=False),
        scratch_types=[
            pltpu.VMEM_SHARED((_NP, _D), jnp.float32),
            pltpu.VMEM_SHARED((_NP, _L), jnp.float32),
            pltpu.VMEM((_CE,), jnp.int32),
            pltpu.VMEM((_CE,), jnp.int32),
            pltpu.VMEM((_CE,), jnp.int32),
            pltpu.VMEM((_CE,), jnp.int32),
            pltpu.VMEM((_CE, _D), jnp.float32),
            pltpu.VMEM((_CE, _D), jnp.float32),
            pltpu.VMEM((_CE, _D), jnp.float32),
            pltpu.VMEM((_CE, _D), jnp.float32),
            pltpu.VMEM((_CE, _L), jnp.float32),
            pltpu.VMEM((_CE, _L), jnp.float32),
            pltpu.VMEM((_D,), jnp.float32),
            pltpu.SemaphoreType.DMA,
            pltpu.SemaphoreType.DMA,
            pltpu.SemaphoreType.DMA,
            pltpu.SemaphoreType.DMA,
            pltpu.SemaphoreType.DMA,
            pltpu.SemaphoreType.DMA,
        ],
    )
    def edge_kernel(xl_hbm, xr_hbm, src_hbm, dst_hbm, att_hbm,
                    num_out, den_out,
                    num_sh, den_sh, srcv0, srcv1, dstv0, dstv1,
                    xlv0, xlv1, xrv0, xrv1, sv0, sv1, attv,
                    semi0, semi1, semd0, semd1, semsc0, semsc1):
        cid = lax.axis_index("c")
        sid = lax.axis_index("s")
        wid = sid * _NC + cid
        lanes = lax.iota(jnp.int32, _L)
        zero16 = jnp.zeros((_L,), jnp.float32)
        zero16i = jnp.zeros((_L,), jnp.int32)
        srcvs = (srcv0, srcv1)
        dstvs = (dstv0, dstv1)
        xlvs = (xlv0, xlv1)
        xrvs = (xrv0, xrv1)
        svs = (sv0, sv1)
        semis = (semi0, semi1)
        semds = (semd0, semd1)
        semscs = (semsc0, semsc1)

        def _zn(i, carry):
            for k in range(_D // _L):
                xlv0[i, pl.ds(k * _L, _L)] = zero16
            return carry
        lax.fori_loop(0, _RZ, _zn, 0)

        def _zs(i, carry):
            sv0[i, :] = zero16
            return carry
        lax.fori_loop(0, _CE, _zs, 0)

        row0 = sid * _RPS

        def _zinit(j, carry):
            r = row0 + j * _RZ
            pltpu.sync_copy(xlv0, num_sh.at[pl.ds(r, _RZ)])
            pltpu.sync_copy(sv0, den_sh.at[pl.ds(r, _RZ)])
            return carry
        lax.fori_loop(0, _RPS // _RZ, _zinit, 0)
        pltpu.sync_copy(att_hbm, attv)
        plsc.subcore_barrier()

        base0 = wid * _EW

        def _start_idx(t, b):
            off = base0 + t * _CE
            return (
                pltpu.async_copy(src_hbm.at[pl.ds(off, _CE)], srcvs[b],
                                 semis[b]),
                pltpu.async_copy(dst_hbm.at[pl.ds(off, _CE)], dstvs[b],
                                 semis[b]),
            )

        def _wait_idx(b):
            pltpu.make_async_copy(src_hbm.at[pl.ds(0, _CE)], srcvs[b],
                                  semis[b]).wait()
            pltpu.make_async_copy(dst_hbm.at[pl.ds(0, _CE)], dstvs[b],
                                  semis[b]).wait()

        def _start_rows(b):
            pltpu.async_copy(xl_hbm.at[srcvs[b]], xlvs[b], semds[b])
            pltpu.async_copy(xr_hbm.at[dstvs[b]], xrvs[b], semds[b])

        def _wait_rows(b):
            pltpu.make_async_copy(xl_hbm.at[srcvs[b]], xlvs[b],
                                  semds[b]).wait()
            pltpu.make_async_copy(xr_hbm.at[dstvs[b]], xrvs[b],
                                  semds[b]).wait()

        _start_idx(0, 0)
        _wait_idx(0)
        _start_rows(0)
        _start_idx(1, 1)

        ngrp = _CE // _L
        def _compute(t, b, attks):
            xlv = xlvs[b]
            xrv = xrvs[b]
            sv = svs[b]
            base = base0 + t * _CE

            def _edge(e, masked):
                parts = []
                for k in range(_D // _L):
                    z = (xlv[e, pl.ds(k * _L, _L)]
                         + xrv[e, pl.ds(k * _L, _L)])
                    m = jnp.maximum(z, _NEG * z)
                    parts.append(m * attks[k])
                while len(parts) > 1:
                    parts = [parts[i] + parts[i + 1]
                             for i in range(0, len(parts), 2)]
                alpha = jnp.sum(parts[0])
                ev = jnp.exp(jnp.full((_L,), alpha, jnp.float32))
                if masked:
                    valid = base + e < _ETOT
                    sv[e, :] = jnp.where(
                        jnp.logical_and(lanes == 0, valid), ev, zero16)
                    sev = jnp.where(valid, ev, zero16)
                else:
                    sv[e, :] = jnp.where(lanes == 0, ev, zero16)
                    sev = ev
                for k in range(_D // _L):
                    xlv[e, pl.ds(k * _L, _L)] = (
                        xlv[e, pl.ds(k * _L, _L)] * sev)

            def _eloop(masked):
                @plsc.parallel_loop(0, _CE, 1, unroll=2)
                def _(e):
                    _edge(e, masked)

            @pl.when(base + _CE <= _ETOT)
            def _():
                _eloop(False)

            @pl.when(base + _CE > _ETOT)
            def _():
                _eloop(True)

            pltpu.async_copy(xlv, num_sh.at[dstvs[b]], semscs[b], add=True)
            pltpu.async_copy(sv, den_sh.at[dstvs[b]], semscs[b], add=True)

        def _wait_scatter(b):
            pltpu.make_async_copy(xlvs[b], num_sh.at[dstvs[b]],
                                  semscs[b]).wait()
            pltpu.make_async_copy(svs[b], den_sh.at[dstvs[b]],
                                  semscs[b]).wait()

        attks = [attv[pl.ds(k * _L, _L)] for k in range(_D // _L)]

        def pair_body(tt, carry):
            for b in range(2):
                t = tt * 2 + b
                nb = 1 - b

                @pl.when(t + 1 < _CHUNKS)
                def _():
                    _wait_idx(nb)

                    @pl.when(t >= 1)
                    def _():
                        _wait_scatter(nb)

                    _start_rows(nb)

                _wait_rows(b)
                _compute(t, b, attks)

                @pl.when(t + 2 < _CHUNKS)
                def _():
                    _start_idx(t + 2, b)
            return carry

        lax.fori_loop(0, _CHUNKS // 2, pair_body, 0)
        _wait_scatter(0)
        _wait_scatter(1)
        plsc.subcore_barrier()

        def _wb(j, carry):
            r = row0 + j * _RZ
            pltpu.sync_copy(num_sh.at[pl.ds(r, _RZ)], xlv0)
            pltpu.sync_copy(xlv0, num_out.at[cid, pl.ds(r, _RZ)])
            pltpu.sync_copy(den_sh.at[pl.ds(r, _RZ)], sv0)
            pltpu.sync_copy(sv0, den_out.at[cid, pl.ds(r, _RZ)])
            return carry
        lax.fori_loop(0, _RPS // _RZ, _wb, 0)

    return edge_kernel


_edge_pass = _build_edge_pass()


_BR = 1000


def _mm1_body(x_ref, wl_ref, bl_ref, wr_ref, br_ref, xl_ref, xr_ref):
    xb = x_ref[...]
    xl_ref[...] = jnp.dot(xb, wl_ref[...], precision=lax.Precision.HIGHEST,
                          preferred_element_type=jnp.float32) + bl_ref[...]
    xr_ref[...] = jnp.dot(xb, wr_ref[...], precision=lax.Precision.HIGHEST,
                          preferred_element_type=jnp.float32) + br_ref[...]


def _proj1(x, wl, bl, wr, br):
    return pl.pallas_call(
        _mm1_body,
        grid=(_N // _BR,),
        in_specs=[
            pl.BlockSpec((_BR, _D), lambda i: (i, 0)),
            pl.BlockSpec((_D, _D), lambda i: (0, 0)),
            pl.BlockSpec((1, _D), lambda i: (0, 0)),
            pl.BlockSpec((_D, _D), lambda i: (0, 0)),
            pl.BlockSpec((1, _D), lambda i: (0, 0)),
        ],
        out_specs=[pl.BlockSpec((_BR, _D), lambda i: (i, 0)),
                   pl.BlockSpec((_BR, _D), lambda i: (i, 0))],
        out_shape=[jax.ShapeDtypeStruct((_N, _D), jnp.float32)] * 2,
    )(x, wl, bl.reshape(1, _D), wr, br.reshape(1, _D))


def _combine_mm_body(num_ref0, num_ref1, den_ref0, den_ref1, bias_ref,
                     wl_ref, bl_ref, wr_ref, br_ref, xl_ref, xr_ref):
    den = den_ref0[0, :, 0:1] + den_ref1[0, :, 0:1] + 1e-16
    h = (num_ref0[0] + num_ref1[0]) / den + bias_ref[...]
    h = jnp.maximum(h, 0.0)
    xl_ref[...] = jnp.dot(h, wl_ref[...], precision=lax.Precision.HIGHEST,
                          preferred_element_type=jnp.float32) + bl_ref[...]
    xr_ref[...] = jnp.dot(h, wr_ref[...], precision=lax.Precision.HIGHEST,
                          preferred_element_type=jnp.float32) + br_ref[...]


def _proj2(num, den, bias, wl, bl, wr, br):
    return pl.pallas_call(
        _combine_mm_body,
        grid=(_N // _BR,),
        in_specs=[
            pl.BlockSpec((1, _BR, _D), lambda i: (0, i, 0)),
            pl.BlockSpec((1, _BR, _D), lambda i: (1, i, 0)),
            pl.BlockSpec((1, _BR, _L), lambda i: (0, i, 0)),
            pl.BlockSpec((1, _BR, _L), lambda i: (1, i, 0)),
            pl.BlockSpec((1, _D), lambda i: (0, 0)),
            pl.BlockSpec((_D, _D), lambda i: (0, 0)),
            pl.BlockSpec((1, _D), lambda i: (0, 0)),
            pl.BlockSpec((_D, _D), lambda i: (0, 0)),
            pl.BlockSpec((1, _D), lambda i: (0, 0)),
        ],
        out_specs=[pl.BlockSpec((_BR, _D), lambda i: (i, 0)),
                   pl.BlockSpec((_BR, _D), lambda i: (i, 0))],
        out_shape=[jax.ShapeDtypeStruct((_N, _D), jnp.float32)] * 2,
    )(num, num, den, den, bias.reshape(1, _D), wl, bl.reshape(1, _D),
      wr, br.reshape(1, _D))


def _final_body(num_ref0, num_ref1, den_ref0, den_ref1, bias_ref, out_ref):
    den = den_ref0[0, :, 0:1] + den_ref1[0, :, 0:1] + 1e-16
    h = (num_ref0[0] + num_ref1[0]) / den + bias_ref[...]
    out_ref[...] = jnp.maximum(h, 0.0)


def _final(num, den, bias):
    return pl.pallas_call(
        _final_body,
        grid=(_N // _BR,),
        in_specs=[
            pl.BlockSpec((1, _BR, _D), lambda i: (0, i, 0)),
            pl.BlockSpec((1, _BR, _D), lambda i: (1, i, 0)),
            pl.BlockSpec((1, _BR, _L), lambda i: (0, i, 0)),
            pl.BlockSpec((1, _BR, _L), lambda i: (1, i, 0)),
            pl.BlockSpec((1, _D), lambda i: (0, 0)),
        ],
        out_specs=pl.BlockSpec((_BR, _D), lambda i: (i, 0)),
        out_shape=jax.ShapeDtypeStruct((_N, _D), jnp.float32),
    )(num, num, den, den, bias.reshape(1, _D))


def kernel(x, edge_index, W_l1, b_l1, W_r1, b_r1, att1, bias1,
           W_l2, b_l2, W_r2, b_r2, att2, bias2):
    idt = edge_index.dtype
    loop = jnp.arange(_N, dtype=idt)
    padz = jnp.zeros((_EPAD - _ETOT,), dtype=idt)
    src = jnp.concatenate([edge_index[0], loop, padz])
    dst = jnp.concatenate([edge_index[1], loop, padz])

    xl1, xr1 = _proj1(x, W_l1, b_l1, W_r1, b_r1)
    num1, den1 = _edge_pass(xl1, xr1, src, dst, att1.reshape(_D))
    xl2, xr2 = _proj2(num1, den1, bias1, W_l2, b_l2, W_r2, b_r2)
    num2, den2 = _edge_pass(xl2, xr2, src, dst, att2.reshape(_D))
    return _final(num2, den2, bias2)
